# baseline (speedup 1.0000x reference)
import jax
import jax.numpy as jnp
from jax import lax
from jax.experimental import pallas as pl
from jax.experimental.pallas import tpu as pltpu

N = 5000
BR = 128
NB = 40
NP = NB * BR
QC = 5120
NQ = 1
KP = 1024
MAXN = 1000
RADIUS = 0.1


def _fused_body(*refs):
    dq_refs = refs[:NQ]
    srow_ref, sblk_ref, out_ref, vrow_s, accr_s, accc_s, spad_s = refs[NQ:]
    i = pl.program_id(0)

    @pl.when(i == 0)
    def _init():
        accc_s[...] = jnp.zeros((1, NP), jnp.float32)
        spad_s[...] = jnp.concatenate(
            [srow_ref[...], jnp.full((1, NP - N), -1.0, jnp.float32)],
            axis=1)

    @pl.when(i < NB)
    def _scan():
        sblk = sblk_ref[...]
        scol = jnp.transpose(sblk, (1, 0))
        m = jnp.full((BR, 1), -1.0, jnp.float32)
        for q, dq in enumerate(dq_refs):
            tq = jnp.where(dq[...] < RADIUS,
                           spad_s[:, q * QC:(q + 1) * QC], -1.0)
            m = jnp.maximum(m, jnp.max(tq, axis=1, keepdims=True))
        gr = i * BR + lax.broadcasted_iota(jnp.int32, (BR, 1), 0)
        keep = (scol >= m) & (gr < N)
        vb = jnp.where(keep, scol, jnp.where(gr < N, 0.0, -1.0))
        vbr = jnp.transpose(vb, (1, 0))
        vrow_s[:, pl.ds(i * BR, BR)] = vbr

        jt = lax.broadcasted_iota(jnp.int32, (BR, BR), 1)
        it = lax.broadcasted_iota(jnp.int32, (BR, BR), 0)
        diag = ((vbr > vb) | ((vbr == vb) & (jt < it))).astype(jnp.float32)

        def body4(c, gs):
            va = vrow_s[:, pl.ds(c * 4 * BR, 4 * BR)]
            g = (va >= vb).astype(jnp.float32)
            cs = jnp.sum(g, axis=0, keepdims=True)
            accc_s[:, pl.ds(c * 4 * BR, 4 * BR)] += float(BR) - cs
            return (gs + g[:, :BR] + g[:, BR:2 * BR]
                    + g[:, 2 * BR:3 * BR] + g[:, 3 * BR:])

        def body1(k, gs):
            va = vrow_s[:, pl.ds(k * BR, BR)]
            g = (va >= vb).astype(jnp.float32)
            cs = jnp.sum(g, axis=0, keepdims=True)
            accc_s[:, pl.ds(k * BR, BR)] += float(BR) - cs
            return gs + g

        nc4 = i // 4
        gs = lax.fori_loop(0, nc4, body4, diag)
        gsum = lax.fori_loop(nc4 * 4, i, body1, gs)
        accr_s[pl.ds(i * BR, BR), :] = jnp.sum(gsum, axis=1, keepdims=True)

    @pl.when(i == NB)
    def _select():
        acct = jnp.transpose(accc_s[...], (1, 0))
        rank = (accr_s[...] + acct).astype(jnp.int32)
        sel_r = 640
        kio = lax.broadcasted_iota(jnp.int32, (sel_r, KP), 1)
        isub = lax.broadcasted_iota(jnp.int32, (sel_r, KP), 0)
        acc = jnp.zeros((1, KP), jnp.int32)
        for t in range(NP // sel_r):
            blk = rank[t * sel_r:(t + 1) * sel_r]
            hitv = jnp.where(blk == kio, isub + t * sel_r, 0)
            acc = acc + jnp.sum(hitv, axis=0, keepdims=True)
        out_ref[...] = acc


def kernel(dists, scores, max_num):
    del max_num
    d2 = dists.reshape(N, N)
    srow = scores.reshape(1, N)

    picked = pl.pallas_call(
        _fused_body,
        grid=(NB + 1,),
        in_specs=[
            pl.BlockSpec((BR, QC),
                         lambda i, q=q: (jnp.minimum(i, NB - 1), q))
            for q in range(NQ)
        ] + [
            pl.BlockSpec((1, N), lambda i: (0, 0)),
            pl.BlockSpec((1, BR), lambda i: (0, jnp.minimum(i, NB - 1))),
        ],
        out_specs=pl.BlockSpec((1, KP), lambda i: (0, 0)),
        out_shape=jax.ShapeDtypeStruct((1, KP), jnp.int32),
        scratch_shapes=[
            pltpu.VMEM((1, NP), jnp.float32),
            pltpu.VMEM((NP, 1), jnp.float32),
            pltpu.VMEM((1, NP), jnp.float32),
            pltpu.VMEM((1, NP), jnp.float32),
        ],
    )(*([d2] * NQ), srow, srow)

    return picked[:, :MAXN]


# speedup vs baseline: 1.0193x; 1.0193x over previous
import jax
import jax.numpy as jnp
from jax import lax
from jax.experimental import pallas as pl
from jax.experimental.pallas import tpu as pltpu

N = 5000
BR = 128
NB = 40
NP = NB * BR
QC = 640
NQ = 8
KP = 1024
MAXN = 1000
RADIUS = 0.1


def _fused_body(*refs):
    dq_refs = refs[:NQ]
    srow_ref, sblk_ref, out_ref, vrow_s, accr_s, accc_s, spad_s = refs[NQ:]
    i = pl.program_id(0)

    @pl.when(i == 0)
    def _init():
        accc_s[...] = jnp.zeros((1, NP), jnp.float32)
        spad_s[...] = jnp.concatenate(
            [srow_ref[...], jnp.full((1, NP - N), -1.0, jnp.float32)],
            axis=1)

    @pl.when(i < NB)
    def _scan():
        sblk = sblk_ref[...]
        scol = jnp.transpose(sblk, (1, 0))
        m = jnp.full((BR, 1), -1.0, jnp.float32)
        for q, dq in enumerate(dq_refs):
            tq = jnp.where(dq[...] < RADIUS,
                           spad_s[:, q * QC:(q + 1) * QC], -1.0)
            m = jnp.maximum(m, jnp.max(tq, axis=1, keepdims=True))
        gr = i * BR + lax.broadcasted_iota(jnp.int32, (BR, 1), 0)
        keep = (scol >= m) & (gr < N)
        vb = jnp.where(keep, scol, jnp.where(gr < N, 0.0, -1.0))
        vbr = jnp.transpose(vb, (1, 0))
        vrow_s[:, pl.ds(i * BR, BR)] = vbr

        jt = lax.broadcasted_iota(jnp.int32, (BR, BR), 1)
        it = lax.broadcasted_iota(jnp.int32, (BR, BR), 0)
        diag = ((vbr > vb) | ((vbr == vb) & (jt < it))).astype(jnp.float32)

        def body4(c, gs):
            va = vrow_s[:, pl.ds(c * 4 * BR, 4 * BR)]
            g = (va >= vb).astype(jnp.float32)
            cs = jnp.sum(g, axis=0, keepdims=True)
            accc_s[:, pl.ds(c * 4 * BR, 4 * BR)] += float(BR) - cs
            return (gs + g[:, :BR] + g[:, BR:2 * BR]
                    + g[:, 2 * BR:3 * BR] + g[:, 3 * BR:])

        def body1(k, gs):
            va = vrow_s[:, pl.ds(k * BR, BR)]
            g = (va >= vb).astype(jnp.float32)
            cs = jnp.sum(g, axis=0, keepdims=True)
            accc_s[:, pl.ds(k * BR, BR)] += float(BR) - cs
            return gs + g

        nc4 = i // 4
        gs = lax.fori_loop(0, nc4, body4, diag)
        gsum = lax.fori_loop(nc4 * 4, i, body1, gs)
        accr_s[pl.ds(i * BR, BR), :] = jnp.sum(gsum, axis=1, keepdims=True)

    @pl.when(i == NB)
    def _select():
        acct = jnp.transpose(accc_s[...], (1, 0))
        rank = (accr_s[...] + acct).astype(jnp.int32)
        sel_r = 640
        kio = lax.broadcasted_iota(jnp.int32, (sel_r, KP), 1)
        isub = lax.broadcasted_iota(jnp.int32, (sel_r, KP), 0)
        acc = jnp.zeros((1, KP), jnp.int32)
        for t in range(NP // sel_r):
            blk = rank[t * sel_r:(t + 1) * sel_r]
            hitv = jnp.where(blk == kio, isub + t * sel_r, 0)
            acc = acc + jnp.sum(hitv, axis=0, keepdims=True)
        out_ref[...] = acc


def kernel(dists, scores, max_num):
    del max_num
    d2 = dists.reshape(N, N)
    srow = scores.reshape(1, N)

    picked = pl.pallas_call(
        _fused_body,
        grid=(NB + 1,),
        in_specs=[
            pl.BlockSpec((BR, QC),
                         lambda i, q=q: (jnp.minimum(i, NB - 1), q))
            for q in range(NQ)
        ] + [
            pl.BlockSpec((1, N), lambda i: (0, 0)),
            pl.BlockSpec((1, BR), lambda i: (0, jnp.minimum(i, NB - 1))),
        ],
        out_specs=pl.BlockSpec((1, KP), lambda i: (0, 0)),
        out_shape=jax.ShapeDtypeStruct((1, KP), jnp.int32),
        scratch_shapes=[
            pltpu.VMEM((1, NP), jnp.float32),
            pltpu.VMEM((NP, 1), jnp.float32),
            pltpu.VMEM((1, NP), jnp.float32),
            pltpu.VMEM((1, NP), jnp.float32),
        ],
    )(*([d2] * NQ), srow, srow)

    return picked[:, :MAXN]
